# baseline (speedup 1.0000x reference)
import functools

import jax
import jax.numpy as jnp
from jax import lax
from jax.experimental import pallas as pl
from jax.experimental.pallas import tpu as pltpu
from jax.experimental.pallas import tpu_sc as plsc

B = 1024
N = 4096
M = 16384
E = 16384
L = 16
EG = E // L
MG = M // L


def _sc_body(x_hbm, pk_hbm, val_hbm, out_hbm,
             pk_v, vals_v,
             acc00, acc01, acc10, acc11,
             xb00, xb01, xb10, xb11,
             xsem0, xsem1, osem0, osem1):
    info = plsc.get_sparse_core_info()
    nc = info.num_cores
    per_tile = B // (nc * info.num_subcores)
    wid = lax.axis_index("s") * nc + lax.axis_index("c")
    b_base = wid * per_tile

    pltpu.sync_copy(pk_hbm, pk_v)
    pltpu.sync_copy(val_hbm, vals_v)

    accs = ((acc00, acc01), (acc10, acc11))
    xbs = ((xb00, xb01), (xb10, xb11))
    xsems = (xsem0, xsem1)
    osems = (osem0, osem1)
    n_pairs = per_tile // 2

    pltpu.async_copy(x_hbm.at[b_base], xb00, xsem0)
    pltpu.async_copy(x_hbm.at[b_base + 1], xb01, xsem0)

    for p in range(n_pairs):
        par = p & 1
        a0, a1 = accs[par]
        x0, x1 = xbs[par]
        b0 = b_base + 2 * p

        if p + 1 < n_pairs:
            nxt = b_base + 2 * (p + 1)
            pltpu.async_copy(x_hbm.at[nxt], xbs[1 - par][0], xsems[1 - par])
            pltpu.async_copy(x_hbm.at[nxt + 1], xbs[1 - par][1], xsems[1 - par])

        if p >= 2:
            prv = b_base + 2 * (p - 2)
            pltpu.make_async_copy(a0, out_hbm.at[prv], osems[par]).wait()
            pltpu.make_async_copy(a1, out_hbm.at[prv + 1], osems[par]).wait()

        @plsc.parallel_loop(0, MG, unroll=8)
        def _zero(k):
            s = pl.ds(k * L, L)
            z = jnp.zeros((L,), jnp.float32)
            a0[s] = z
            a1[s] = z

        pltpu.make_async_copy(x_hbm.at[b0], x0, xsems[par]).wait()
        pltpu.make_async_copy(x_hbm.at[b0 + 1], x1, xsems[par]).wait()

        @plsc.parallel_loop(0, EG, unroll=8)
        def _entry(g):
            s = pl.ds(g * L, L)
            pk = pk_v[s]
            v = vals_v[s]
            r = pk >> 12
            c = pk & 4095
            plsc.addupdate_scatter(a0, [r], plsc.load_gather(x0, [c]) * v)
            plsc.addupdate_scatter(a1, [r], plsc.load_gather(x1, [c]) * v)

        pltpu.async_copy(a0, out_hbm.at[b0], osems[par])
        pltpu.async_copy(a1, out_hbm.at[b0 + 1], osems[par])

    for p in (n_pairs - 2, n_pairs - 1):
        par = p & 1
        a0, a1 = accs[par]
        b0 = b_base + 2 * p
        pltpu.make_async_copy(a0, out_hbm.at[b0], osems[par]).wait()
        pltpu.make_async_copy(a1, out_hbm.at[b0 + 1], osems[par]).wait()


@jax.jit
def _sc_spmm(x, packed, vals):
    mesh = plsc.VectorSubcoreMesh(core_axis_name="c", subcore_axis_name="s")
    kfn = functools.partial(
        pl.kernel,
        out_type=jax.ShapeDtypeStruct((B, M), jnp.float32),
        mesh=mesh,
        compiler_params=pltpu.CompilerParams(needs_layout_passes=False),
        scratch_types=[
            pltpu.VMEM((E,), jnp.int32),
            pltpu.VMEM((E,), jnp.float32),
            pltpu.VMEM((M,), jnp.float32),
            pltpu.VMEM((M,), jnp.float32),
            pltpu.VMEM((M,), jnp.float32),
            pltpu.VMEM((M,), jnp.float32),
            pltpu.VMEM((N,), jnp.float32),
            pltpu.VMEM((N,), jnp.float32),
            pltpu.VMEM((N,), jnp.float32),
            pltpu.VMEM((N,), jnp.float32),
            pltpu.SemaphoreType.DMA,
            pltpu.SemaphoreType.DMA,
            pltpu.SemaphoreType.DMA,
            pltpu.SemaphoreType.DMA,
        ],
    )(_sc_body)
    return kfn(x, packed, vals)


def kernel(input, weight_indices, weight_values):
    wi = weight_indices.astype(jnp.int32)
    packed = wi[0] * 4096 + wi[1]
    return _sc_spmm(input, packed, weight_values)


# speedup vs baseline: 1.1438x; 1.1438x over previous
import functools

import jax
import jax.numpy as jnp
from jax import lax
from jax.experimental import pallas as pl
from jax.experimental.pallas import tpu as pltpu
from jax.experimental.pallas import tpu_sc as plsc

B = 1024
N = 4096
M = 16384
E = 16384
L = 16
EG = E // L
MG = M // L


def _sc_body(x_hbm, pk_hbm, val_hbm, out_hbm,
             pk_v, vals_v,
             acc00, acc01, acc10, acc11,
             xb00, xb01, xb10, xb11,
             xsem0, xsem1, osem0, osem1):
    info = plsc.get_sparse_core_info()
    nc = info.num_cores
    per_tile = B // (nc * info.num_subcores)
    wid = lax.axis_index("s") * nc + lax.axis_index("c")
    b_base = wid * per_tile

    pltpu.sync_copy(pk_hbm, pk_v)
    pltpu.sync_copy(val_hbm, vals_v)

    accs = ((acc00, acc01), (acc10, acc11))
    xbs = ((xb00, xb01), (xb10, xb11))
    xsems = (xsem0, xsem1)
    osems = (osem0, osem1)
    n_pairs = per_tile // 2

    pltpu.async_copy(x_hbm.at[b_base], xb00, xsem0)
    pltpu.async_copy(x_hbm.at[b_base + 1], xb01, xsem0)

    @plsc.parallel_loop(0, MG, unroll=8)
    def _zero0(k):
        s = pl.ds(k * L, L)
        z = jnp.zeros((L,), jnp.float32)
        acc00[s] = z
        acc01[s] = z

    HALF = EG // 2

    for p in range(n_pairs):
        par = p & 1
        a0, a1 = accs[par]
        o0, o1 = accs[1 - par]
        x0, x1 = xbs[par]
        b0 = b_base + 2 * p

        if p + 1 < n_pairs:
            nxt = b_base + 2 * (p + 1)
            pltpu.async_copy(x_hbm.at[nxt], xbs[1 - par][0], xsems[1 - par])
            pltpu.async_copy(x_hbm.at[nxt + 1], xbs[1 - par][1], xsems[1 - par])

        pltpu.make_async_copy(x_hbm.at[b0], x0, xsems[par]).wait()
        pltpu.make_async_copy(x_hbm.at[b0 + 1], x1, xsems[par]).wait()

        @plsc.parallel_loop(0, HALF, unroll=4)
        def _entry_a(g):
            s = pl.ds(g * L, L)
            pk = pk_v[s]
            v = vals_v[s]
            r = pk >> 12
            c = pk & 4095
            plsc.addupdate_scatter(a0, [r], plsc.load_gather(x0, [c]) * v)
            plsc.addupdate_scatter(a1, [r], plsc.load_gather(x1, [c]) * v)

        if p >= 1:
            prv = b_base + 2 * (p - 1)
            pltpu.make_async_copy(o0, out_hbm.at[prv], osems[1 - par]).wait()
            pltpu.make_async_copy(o1, out_hbm.at[prv + 1], osems[1 - par]).wait()

        @plsc.parallel_loop(HALF, EG, unroll=4)
        def _entry_b(g):
            s = pl.ds(g * L, L)
            pk = pk_v[s]
            v = vals_v[s]
            r = pk >> 12
            c = pk & 4095
            plsc.addupdate_scatter(a0, [r], plsc.load_gather(x0, [c]) * v)
            plsc.addupdate_scatter(a1, [r], plsc.load_gather(x1, [c]) * v)
            z = jnp.zeros((L,), jnp.float32)
            k = (g - HALF) * 2
            o0[pl.ds(k * L, L)] = z
            o0[pl.ds((k + 1) * L, L)] = z
            o1[pl.ds(k * L, L)] = z
            o1[pl.ds((k + 1) * L, L)] = z

        pltpu.async_copy(a0, out_hbm.at[b0], osems[par])
        pltpu.async_copy(a1, out_hbm.at[b0 + 1], osems[par])

    p = n_pairs - 1
    par = p & 1
    a0, a1 = accs[par]
    b0 = b_base + 2 * p
    pltpu.make_async_copy(a0, out_hbm.at[b0], osems[par]).wait()
    pltpu.make_async_copy(a1, out_hbm.at[b0 + 1], osems[par]).wait()


@jax.jit
def _sc_spmm(x, packed, vals):
    mesh = plsc.VectorSubcoreMesh(core_axis_name="c", subcore_axis_name="s")
    kfn = functools.partial(
        pl.kernel,
        out_type=jax.ShapeDtypeStruct((B, M), jnp.float32),
        mesh=mesh,
        compiler_params=pltpu.CompilerParams(needs_layout_passes=False),
        scratch_types=[
            pltpu.VMEM((E,), jnp.int32),
            pltpu.VMEM((E,), jnp.float32),
            pltpu.VMEM((M,), jnp.float32),
            pltpu.VMEM((M,), jnp.float32),
            pltpu.VMEM((M,), jnp.float32),
            pltpu.VMEM((M,), jnp.float32),
            pltpu.VMEM((N,), jnp.float32),
            pltpu.VMEM((N,), jnp.float32),
            pltpu.VMEM((N,), jnp.float32),
            pltpu.VMEM((N,), jnp.float32),
            pltpu.SemaphoreType.DMA,
            pltpu.SemaphoreType.DMA,
            pltpu.SemaphoreType.DMA,
            pltpu.SemaphoreType.DMA,
        ],
    )(_sc_body)
    return kfn(x, packed, vals)


def kernel(input, weight_indices, weight_values):
    wi = weight_indices.astype(jnp.int32)
    packed = wi[0] * 4096 + wi[1]
    return _sc_spmm(input, packed, weight_values)
